# baseline (speedup 1.0000x reference)
import jax
import jax.numpy as jnp
from jax import lax
from jax.experimental import pallas as pl
from jax.experimental.pallas import tpu as pltpu


def _round_up(x, m):
    return ((x + m - 1) // m) * m


def _softplus(x):
    return jnp.logaddexp(x, 0.0)


def _fused_kernel(qp_ref, u_ref, vi_ref, vj_ref, out_ref):
    u = u_ref[...]
    vi = vi_ref[...]
    vj = vj_ref[...]
    D = u.shape[1]

    P = jnp.concatenate([u * vi, u * vj, u * u + vi * vi + vj * vj], axis=1)
    row = lax.broadcasted_iota(jnp.int32, (3 * D, 8), 0)
    col = lax.broadcasted_iota(jnp.int32, (3 * D, 8), 1)
    S = (row // D == col).astype(jnp.float32)
    R = jnp.dot(P, S, preferred_element_type=jnp.float32)

    pop = jnp.tanh(_softplus(qp_ref[...]))
    pred_i = _softplus(R[:, 0:1]) * pop[:, 0:1]
    pred_j = _softplus(R[:, 1:2]) * pop[:, 1:2]
    out_ref[...] = jnp.concatenate([pred_i, pred_j, R[:, 2:8]], axis=1)


def kernel(embed_user, embed_item, q, user, item_i, item_j):
    B = int(user.shape[0])
    D = int(embed_user.shape[1])

    TB = 512
    nt = -(-B // TB)
    Bp = nt * TB
    pad = Bp - B

    def pad_ids(x):
        x = x.astype(jnp.int32)
        return jnp.pad(x, (0, pad)) if pad else x

    u_ids = pad_ids(user)
    i_ids = pad_ids(item_i)
    j_ids = pad_ids(item_j)

    ug = jnp.take(embed_user, u_ids, axis=0).astype(jnp.float32)
    vig = jnp.take(embed_item, i_ids, axis=0).astype(jnp.float32)
    vjg = jnp.take(embed_item, j_ids, axis=0).astype(jnp.float32)
    qp = jnp.stack([jnp.take(q, i_ids), jnp.take(q, j_ids)],
                   axis=1).astype(jnp.float32)

    out = pl.pallas_call(
        _fused_kernel,
        out_shape=jax.ShapeDtypeStruct((Bp, 8), jnp.float32),
        grid=(nt,),
        in_specs=[
            pl.BlockSpec((TB, 2), lambda t: (t, 0)),
            pl.BlockSpec((TB, D), lambda t: (t, 0)),
            pl.BlockSpec((TB, D), lambda t: (t, 0)),
            pl.BlockSpec((TB, D), lambda t: (t, 0)),
        ],
        out_specs=pl.BlockSpec((TB, 8), lambda t: (t, 0)),
        compiler_params=pltpu.CompilerParams(
            dimension_semantics=("parallel",),
            vmem_limit_bytes=64 * 1024 * 1024,
        ),
    )(qp, ug, vig, vjg)

    pred_i = out[:B, 0]
    pred_j = out[:B, 1]
    reg_loss = 0.5 * jnp.sum(out[:B, 2]) / B
    return pred_i, pred_j, reg_loss


# speedup vs baseline: 3.0018x; 3.0018x over previous
import jax
import jax.numpy as jnp
from jax.experimental import pallas as pl
from jax.experimental.pallas import tpu as pltpu

_TB = 512


def _softplus(x):
    return jnp.logaddexp(x, 0.0)


def _user_gather_kernel(ids_ref, tab_ref, out_ref):
    for mi in range(out_ref.shape[0]):
        out_ref[mi, 0] = tab_ref[ids_ref[0, 0, mi], 0]


def _item_compute_kernel(ids_ref, tab_ref, u_ref, out_ref, vi_s, vj_s):
    TB = out_ref.shape[0]
    D = u_ref.shape[2]

    for mi in range(TB):
        vi_s[mi, 0] = tab_ref[ids_ref[0, 0, mi], 0]
        vj_s[mi, 0] = tab_ref[ids_ref[0, 1, mi], 0]

    u = u_ref[...]
    vi = vi_s[:, :, :D]
    vj = vj_s[:, :, :D]
    gi = vi_s[:, :, D:D + 1]
    gj = vj_s[:, :, D:D + 1]

    si = jnp.sum(u * vi, axis=-1, keepdims=True)
    sj = jnp.sum(u * vj, axis=-1, keepdims=True)
    ss = jnp.sum(u * u + vi * vi + vj * vj, axis=-1, keepdims=True)

    pred_i = _softplus(si) * gi
    pred_j = _softplus(sj) * gj
    zeros = jnp.zeros((TB, 1, 5), jnp.float32)
    out_ref[...] = jnp.concatenate([pred_i, pred_j, ss, zeros], axis=-1)


def kernel(embed_user, embed_item, q, user, item_i, item_j):
    B = int(user.shape[0])
    Nu, D = int(embed_user.shape[0]), int(embed_user.shape[1])
    Ni = int(embed_item.shape[0])

    TB = _TB
    nt = -(-B // TB)
    Bp = nt * TB
    pad = Bp - B

    def pad_ids(x):
        x = x.astype(jnp.int32)
        return jnp.pad(x, (0, pad)) if pad else x

    u_ids = pad_ids(user).reshape(nt, 1, TB)
    ij_ids = jnp.stack([pad_ids(item_i).reshape(nt, TB),
                        pad_ids(item_j).reshape(nt, TB)], axis=1)

    ut = embed_user.astype(jnp.float32).reshape(Nu, 1, D)

    g = jnp.tanh(_softplus(q.astype(jnp.float32)))
    it_aug = jnp.concatenate(
        [embed_item.astype(jnp.float32), g[:, None],
         jnp.zeros((Ni, 127), jnp.float32)], axis=1).reshape(Ni, 1, 2 * D)

    cparams = pltpu.CompilerParams(
        dimension_semantics=("parallel",),
        vmem_limit_bytes=57 * 1024 * 1024,
    )
    cparams2 = pltpu.CompilerParams(
        dimension_semantics=("parallel",),
        vmem_limit_bytes=63 * 1024 * 1024,
    )

    ug = pl.pallas_call(
        _user_gather_kernel,
        out_shape=jax.ShapeDtypeStruct((Bp, 1, D), jnp.float32),
        grid=(nt,),
        in_specs=[
            pl.BlockSpec((1, 1, TB), lambda t: (t, 0, 0),
                         memory_space=pltpu.SMEM),
            pl.BlockSpec((Nu, 1, D), lambda t: (0, 0, 0)),
        ],
        out_specs=pl.BlockSpec((TB, 1, D), lambda t: (t, 0, 0)),
        compiler_params=cparams,
    )(u_ids, ut)

    out = pl.pallas_call(
        _item_compute_kernel,
        out_shape=jax.ShapeDtypeStruct((Bp, 1, 8), jnp.float32),
        grid=(nt,),
        in_specs=[
            pl.BlockSpec((1, 2, TB), lambda t: (t, 0, 0),
                         memory_space=pltpu.SMEM),
            pl.BlockSpec((Ni, 1, 2 * D), lambda t: (0, 0, 0)),
            pl.BlockSpec((TB, 1, D), lambda t: (t, 0, 0)),
        ],
        out_specs=pl.BlockSpec((TB, 1, 8), lambda t: (t, 0, 0)),
        scratch_shapes=[
            pltpu.VMEM((TB, 1, 2 * D), jnp.float32),
            pltpu.VMEM((TB, 1, 2 * D), jnp.float32),
        ],
        compiler_params=cparams2,
    )(ij_ids, it_aug, ug)

    o = out.reshape(Bp, 8)
    pred_i = o[:B, 0]
    pred_j = o[:B, 1]
    reg_loss = 0.5 * jnp.sum(o[:B, 2]) / B
    return pred_i, pred_j, reg_loss


# speedup vs baseline: 7.7852x; 2.5935x over previous
import jax
import jax.numpy as jnp
from jax import lax
from jax.experimental import pallas as pl
from jax.experimental.pallas import tpu as pltpu

_TB = 1024
_CB = 2000


def _softplus(x):
    return jnp.logaddexp(x, 0.0)


def _itaug_build_kernel(emb_ref, g_ref, out_ref):
    CB, _, D = emb_ref.shape
    out_ref[:, :, :D] = emb_ref[...]
    out_ref[:, :, D:D + 1] = g_ref[...].reshape(CB, 1, 1)
    out_ref[:, :, D + 1:] = jnp.zeros((CB, 1, D - 1), jnp.float32)


def _user_gather_kernel(ids_ref, tab_ref, out_ref):
    for mi in range(out_ref.shape[0]):
        out_ref[mi] = tab_ref[ids_ref[0, 0, mi], 0]


def _item_compute_kernel(ids_ref, tab_ref, u_ref, out_ref, vi_s, vj_s):
    TB, D = u_ref.shape
    K = 3 * D + 2

    for mi in range(TB):
        vi_s[mi] = tab_ref[ids_ref[0, 0, mi], 0]
        vj_s[mi] = tab_ref[ids_ref[0, 1, mi], 0]

    u = u_ref[...]
    vi = vi_s[:, :D]
    vj = vj_s[:, :D]
    gi = vi_s[:, D:D + 1]
    gj = vj_s[:, D:D + 1]

    P = jnp.concatenate(
        [u * vi, u * vj, u * u + vi * vi + vj * vj, gi, gj], axis=1)
    rowi = lax.broadcasted_iota(jnp.int32, (8, K), 0)
    coli = lax.broadcasted_iota(jnp.int32, (8, K), 1)
    S = (((rowi == coli // D) & (coli < 3 * D))
         | ((rowi == 3) & (coli == 3 * D))
         | ((rowi == 4) & (coli == 3 * D + 1))).astype(jnp.float32)
    R = lax.dot_general(S, P, (((1,), (1,)), ((), ())),
                        preferred_element_type=jnp.float32)

    pred = _softplus(R[0:2, :]) * R[3:5, :]
    out_ref[...] = jnp.concatenate([pred, R[2:3, :]], axis=0)


def kernel(embed_user, embed_item, q, user, item_i, item_j):
    B = int(user.shape[0])
    Nu, D = int(embed_user.shape[0]), int(embed_user.shape[1])
    Ni = int(embed_item.shape[0])

    TB = _TB
    nt = -(-B // TB)
    Bp = nt * TB
    pad = Bp - B

    def pad_ids(x):
        x = x.astype(jnp.int32)
        return jnp.pad(x, (0, pad)) if pad else x

    u_ids = pad_ids(user).reshape(nt, 1, TB)
    ij_ids = jnp.stack([pad_ids(item_i).reshape(nt, TB),
                        pad_ids(item_j).reshape(nt, TB)], axis=1)

    ut = embed_user.astype(jnp.float32).reshape(Nu, 1, D)

    cparams = pltpu.CompilerParams(
        dimension_semantics=("arbitrary",),
        vmem_limit_bytes=57 * 1024 * 1024,
    )
    cparams2 = pltpu.CompilerParams(
        dimension_semantics=("arbitrary",),
        vmem_limit_bytes=63 * 1024 * 1024,
    )

    CB = _CB
    nb = -(-Ni // CB)
    emb3 = embed_item.astype(jnp.float32).reshape(Ni, 1, D)
    g = jnp.tanh(_softplus(q.astype(jnp.float32))).reshape(Ni, 1)
    it_aug = pl.pallas_call(
        _itaug_build_kernel,
        out_shape=jax.ShapeDtypeStruct((Ni, 1, 2 * D), jnp.float32),
        grid=(nb,),
        in_specs=[
            pl.BlockSpec((CB, 1, D), lambda t: (t, 0, 0)),
            pl.BlockSpec((CB, 1), lambda t: (t, 0)),
        ],
        out_specs=pl.BlockSpec((CB, 1, 2 * D), lambda t: (t, 0, 0)),
        compiler_params=pltpu.CompilerParams(
            dimension_semantics=("arbitrary",),
            vmem_limit_bytes=32 * 1024 * 1024,
        ),
    )(emb3, g)

    ug = pl.pallas_call(
        _user_gather_kernel,
        out_shape=jax.ShapeDtypeStruct((Bp, D), jnp.float32),
        grid=(nt,),
        in_specs=[
            pl.BlockSpec((1, 1, TB), lambda t: (t, 0, 0),
                         memory_space=pltpu.SMEM),
            pl.BlockSpec((Nu, 1, D), lambda t: (0, 0, 0)),
        ],
        out_specs=pl.BlockSpec((TB, D), lambda t: (t, 0)),
        compiler_params=cparams,
    )(u_ids, ut)

    out = pl.pallas_call(
        _item_compute_kernel,
        out_shape=jax.ShapeDtypeStruct((3, Bp), jnp.float32),
        grid=(nt,),
        in_specs=[
            pl.BlockSpec((1, 2, TB), lambda t: (t, 0, 0),
                         memory_space=pltpu.SMEM),
            pl.BlockSpec((Ni, 1, 2 * D), lambda t: (0, 0, 0)),
            pl.BlockSpec((TB, D), lambda t: (t, 0)),
        ],
        out_specs=pl.BlockSpec((3, TB), lambda t: (0, t)),
        scratch_shapes=[
            pltpu.VMEM((TB, 2 * D), jnp.float32),
            pltpu.VMEM((TB, 2 * D), jnp.float32),
        ],
        compiler_params=cparams2,
    )(ij_ids, it_aug, ug)

    pred_i = out[0, :B]
    pred_j = out[1, :B]
    reg_loss = 0.5 * jnp.sum(out[2, :B]) / B
    return pred_i, pred_j, reg_loss
